# baseline (speedup 1.0000x reference)
import functools

import jax
import jax.numpy as jnp
from jax.experimental import pallas as pl
from jax.experimental.pallas import tpu as pltpu

_DN = (((1,), (1,)), ((), ()))


def _gcn_body(x_ref, adj_ref, w1_ref, b1_ref, w2_ref, b2_ref, out_ref):
    s1 = jnp.dot(x_ref[...], w1_ref[...], preferred_element_type=jnp.float32)
    h1 = jax.nn.relu(
        jnp.dot(adj_ref[...], s1, preferred_element_type=jnp.float32) + b1_ref[...]
    )
    s2 = jnp.dot(h1, w2_ref[...], preferred_element_type=jnp.float32)
    h2 = jax.nn.relu(
        jnp.dot(adj_ref[...], s2, preferred_element_type=jnp.float32) + b2_ref[...]
    )
    out_ref[...] = h2


def _head_body(nk, flat_ref, fc1w_ref, fc1b_ref, fc2w_ref, fc2b_ref, fc3w_ref,
               fc3b_ref, out_ref, acc_ref):
    k = pl.program_id(0)
    partial = jax.lax.dot_general(flat_ref[...], fc1w_ref[...], _DN,
                                  preferred_element_type=jnp.float32)

    @pl.when(k == 0)
    def _init():
        acc_ref[...] = partial

    @pl.when(k > 0)
    def _accum():
        acc_ref[...] = acc_ref[...] + partial

    @pl.when(k == nk - 1)
    def _finish():
        a1 = jax.nn.relu(acc_ref[...] + fc1b_ref[...])
        a2 = jax.nn.relu(
            jax.lax.dot_general(a1, fc2w_ref[...], _DN,
                                preferred_element_type=jnp.float32) + fc2b_ref[...]
        )
        a3 = jnp.sum(a2 * fc3w_ref[...], axis=1, keepdims=True) + fc3b_ref[...]
        out_ref[...] = jax.nn.sigmoid(a3)


def kernel(x, adj, W1, b1, W2, b2, fc1_w, fc1_b, fc2_w, fc2_b, fc3_w, fc3_b):
    n, nclass = adj.shape[0], W2.shape[1]
    kdim = n * nclass
    chunk = 1024
    nk = kdim // chunk

    h2 = pl.pallas_call(
        _gcn_body,
        out_shape=jax.ShapeDtypeStruct((n, nclass), jnp.float32),
    )(x, adj, W1, b1.reshape(1, -1), W2, b2.reshape(1, -1))

    flat = h2.reshape(1, kdim)
    y = pl.pallas_call(
        functools.partial(_head_body, nk),
        grid=(nk,),
        in_specs=[
            pl.BlockSpec((1, chunk), lambda k: (0, k)),
            pl.BlockSpec((fc1_w.shape[0], chunk), lambda k: (0, k)),
            pl.BlockSpec(fc1_b.reshape(1, -1).shape, lambda k: (0, 0)),
            pl.BlockSpec(fc2_w.shape, lambda k: (0, 0)),
            pl.BlockSpec(fc2_b.reshape(1, -1).shape, lambda k: (0, 0)),
            pl.BlockSpec(fc3_w.shape, lambda k: (0, 0)),
            pl.BlockSpec((1, 1), lambda k: (0, 0)),
        ],
        out_specs=pl.BlockSpec((1, 1), lambda k: (0, 0)),
        out_shape=jax.ShapeDtypeStruct((1, 1), jnp.float32),
        scratch_shapes=[pltpu.VMEM((1, fc1_w.shape[0]), jnp.float32)],
    )(flat, fc1_w, fc1_b.reshape(1, -1), fc2_w, fc2_b.reshape(1, -1),
      fc3_w, fc3_b.reshape(1, -1))

    return y.reshape(1)


# speedup vs baseline: 1.1998x; 1.1998x over previous
import functools

import jax
import jax.numpy as jnp
from jax.experimental import pallas as pl
from jax.experimental.pallas import tpu as pltpu

_DN = (((1,), (1,)), ((), ()))


def _gcn_body(x_ref, adj_ref, w1_ref, b1_ref, w2_ref, b2_ref, out_ref):
    s1 = jnp.dot(x_ref[...], w1_ref[...], preferred_element_type=jnp.float32)
    h1 = jax.nn.relu(
        jnp.dot(adj_ref[...], s1, preferred_element_type=jnp.float32) + b1_ref[...]
    )
    s2 = jnp.dot(h1, w2_ref[...], preferred_element_type=jnp.float32)
    h2 = jax.nn.relu(
        jnp.dot(adj_ref[...], s2, preferred_element_type=jnp.float32) + b2_ref[...]
    )
    out_ref[...] = h2


def _head_body(nk, flat_ref, fc1w_ref, fc1b_ref, fc2w_ref, fc2b_ref, fc3w_ref,
               fc3b_ref, out_ref, acc_ref):
    k = pl.program_id(0)
    partial = jax.lax.dot_general(flat_ref[...], fc1w_ref[...], _DN,
                                  preferred_element_type=jnp.float32)

    @pl.when(k == 0)
    def _init():
        acc_ref[...] = partial

    @pl.when(k > 0)
    def _accum():
        acc_ref[...] = acc_ref[...] + partial

    @pl.when(k == nk - 1)
    def _finish():
        a1 = jax.nn.relu(acc_ref[...] + fc1b_ref[...])
        a2 = jax.nn.relu(
            jax.lax.dot_general(a1, fc2w_ref[...], _DN,
                                preferred_element_type=jnp.float32) + fc2b_ref[...]
        )
        a3 = jnp.sum(a2 * fc3w_ref[...], axis=1, keepdims=True) + fc3b_ref[...]
        out_ref[...] = jax.nn.sigmoid(a3)


def kernel(x, adj, W1, b1, W2, b2, fc1_w, fc1_b, fc2_w, fc2_b, fc3_w, fc3_b):
    n, nclass = adj.shape[0], W2.shape[1]
    kdim = n * nclass
    chunk = 1664
    nk = kdim // chunk

    h2 = pl.pallas_call(
        _gcn_body,
        out_shape=jax.ShapeDtypeStruct((n, nclass), jnp.float32),
    )(x, adj, W1, b1.reshape(1, -1), W2, b2.reshape(1, -1))

    flat = h2.reshape(1, kdim)
    y = pl.pallas_call(
        functools.partial(_head_body, nk),
        grid=(nk,),
        in_specs=[
            pl.BlockSpec((1, chunk), lambda k: (0, k)),
            pl.BlockSpec((fc1_w.shape[0], chunk), lambda k: (0, k)),
            pl.BlockSpec(fc1_b.reshape(1, -1).shape, lambda k: (0, 0)),
            pl.BlockSpec(fc2_w.shape, lambda k: (0, 0)),
            pl.BlockSpec(fc2_b.reshape(1, -1).shape, lambda k: (0, 0)),
            pl.BlockSpec(fc3_w.shape, lambda k: (0, 0)),
            pl.BlockSpec((1, 1), lambda k: (0, 0)),
        ],
        out_specs=pl.BlockSpec((1, 1), lambda k: (0, 0)),
        out_shape=jax.ShapeDtypeStruct((1, 1), jnp.float32),
        scratch_shapes=[pltpu.VMEM((1, fc1_w.shape[0]), jnp.float32)],
    )(flat, fc1_w, fc1_b.reshape(1, -1), fc2_w, fc2_b.reshape(1, -1),
      fc3_w, fc3_b.reshape(1, -1))

    return y.reshape(1)


# speedup vs baseline: 2.1273x; 1.7730x over previous
import jax
import jax.numpy as jnp
from jax.experimental import pallas as pl
from jax.experimental.pallas import tpu as pltpu

_DN = (((1,), (1,)), ((), ()))
_NCHUNK = 13


def _body(x_ref, adj_ref, w1_ref, b1_ref, w2_ref, b2_ref, fc1w_hbm,
          fc1b_ref, fc2w_ref, fc2b_ref, fc3w_ref, fc3b_ref, out_ref,
          wbuf, flat_s, sems):
    nout, kdim = wbuf.shape
    chunk = kdim // _NCHUNK
    for k in range(_NCHUNK):
        sl = pl.ds(k * chunk, chunk)
        pltpu.make_async_copy(fc1w_hbm.at[:, sl], wbuf.at[:, sl],
                              sems.at[k]).start()

    s1 = jnp.dot(x_ref[...], w1_ref[...], preferred_element_type=jnp.float32)
    h1 = jax.nn.relu(
        jnp.dot(adj_ref[...], s1, preferred_element_type=jnp.float32) + b1_ref[...]
    )
    s2 = jnp.dot(h1, w2_ref[...], preferred_element_type=jnp.float32)
    h2 = jax.nn.relu(
        jnp.dot(adj_ref[...], s2, preferred_element_type=jnp.float32) + b2_ref[...]
    )
    n, nclass = h2.shape
    for r in range(n):
        flat_s[0:1, r * nclass:(r + 1) * nclass] = h2[r:r + 1, :]

    acc = jnp.zeros((nout, 128), jnp.float32)
    for k in range(_NCHUNK):
        sl = pl.ds(k * chunk, chunk)
        pltpu.make_async_copy(fc1w_hbm.at[:, sl], wbuf.at[:, sl],
                              sems.at[k]).wait()
        t = wbuf[:, sl] * flat_s[0:1, k * chunk:(k + 1) * chunk]
        for g in range(chunk // 128):
            acc = acc + t[:, g * 128:(g + 1) * 128]

    a1 = jax.nn.relu(acc.sum(axis=1).reshape(1, nout) + fc1b_ref[...])
    a2 = jax.nn.relu(
        jax.lax.dot_general(a1, fc2w_ref[...], _DN,
                            preferred_element_type=jnp.float32) + fc2b_ref[...]
    )
    a3 = jnp.sum(a2 * fc3w_ref[...], axis=1, keepdims=True) + fc3b_ref[...]
    out_ref[...] = jax.nn.sigmoid(a3)


def kernel(x, adj, W1, b1, W2, b2, fc1_w, fc1_b, fc2_w, fc2_b, fc3_w, fc3_b):
    nout, kdim = fc1_w.shape
    vmem = pl.BlockSpec(memory_space=pltpu.MemorySpace.VMEM)

    y = pl.pallas_call(
        _body,
        in_specs=[vmem, vmem, vmem, vmem, vmem, vmem,
                  pl.BlockSpec(memory_space=pl.ANY),
                  vmem, vmem, vmem, vmem, vmem],
        out_shape=jax.ShapeDtypeStruct((1, 1), jnp.float32),
        scratch_shapes=[
            pltpu.VMEM((nout, kdim), jnp.float32),
            pltpu.VMEM((1, kdim), jnp.float32),
            pltpu.SemaphoreType.DMA((_NCHUNK,)),
        ],
    )(x, adj, W1, b1.reshape(1, -1), W2, b2.reshape(1, -1), fc1_w,
      fc1_b.reshape(1, -1), fc2_w, fc2_b.reshape(1, -1), fc3_w,
      fc3_b.reshape(1, -1))

    return y.reshape(1)


# speedup vs baseline: 2.1401x; 1.0060x over previous
import jax
import jax.numpy as jnp
from jax.experimental import pallas as pl
from jax.experimental.pallas import tpu as pltpu

_DN = (((1,), (1,)), ((), ()))
_NCHUNK = 13


def _body(x_ref, adj_ref, w1_ref, b1_ref, w2_ref, b2_ref, fc1w_hbm,
          fc1b_ref, fc2w_ref, fc2b_ref, fc3w_ref, fc3b_ref, out_ref,
          wbuf, flat_s, sems):
    nout, kdim = wbuf.shape
    chunk = kdim // _NCHUNK
    for k in range(_NCHUNK):
        sl = pl.ds(k * chunk, chunk)
        pltpu.make_async_copy(fc1w_hbm.at[:, sl], wbuf.at[:, sl],
                              sems.at[k]).start()

    s1 = jnp.dot(x_ref[...], w1_ref[...], preferred_element_type=jnp.float32)
    h1 = jax.nn.relu(
        jnp.dot(adj_ref[...], s1, preferred_element_type=jnp.float32)
        + b1_ref[...].reshape(1, -1)
    )
    s2 = jnp.dot(h1, w2_ref[...], preferred_element_type=jnp.float32)
    h2 = jax.nn.relu(
        jnp.dot(adj_ref[...], s2, preferred_element_type=jnp.float32)
        + b2_ref[...].reshape(1, -1)
    )
    n, nclass = h2.shape
    for r in range(n):
        flat_s[0:1, r * nclass:(r + 1) * nclass] = h2[r:r + 1, :]

    acc = jnp.zeros((nout, 128), jnp.float32)
    for k in range(_NCHUNK):
        sl = pl.ds(k * chunk, chunk)
        pltpu.make_async_copy(fc1w_hbm.at[:, sl], wbuf.at[:, sl],
                              sems.at[k]).wait()
        t = wbuf[:, sl] * flat_s[0:1, k * chunk:(k + 1) * chunk]
        for g in range(chunk // 128):
            acc = acc + t[:, g * 128:(g + 1) * 128]

    a1 = jax.nn.relu(acc.sum(axis=1).reshape(1, nout)
                     + fc1b_ref[...].reshape(1, -1))
    a2 = jax.nn.relu(
        jax.lax.dot_general(a1, fc2w_ref[...], _DN,
                            preferred_element_type=jnp.float32)
        + fc2b_ref[...].reshape(1, -1)
    )
    a3 = (jnp.sum(a2 * fc3w_ref[...], axis=1, keepdims=True)
          + fc3b_ref[...].reshape(1, -1))
    out_ref[...] = jax.nn.sigmoid(a3).reshape(1)


def kernel(x, adj, W1, b1, W2, b2, fc1_w, fc1_b, fc2_w, fc2_b, fc3_w, fc3_b):
    nout, kdim = fc1_w.shape
    vmem = pl.BlockSpec(memory_space=pltpu.MemorySpace.VMEM)

    y = pl.pallas_call(
        _body,
        in_specs=[vmem, vmem, vmem, vmem, vmem, vmem,
                  pl.BlockSpec(memory_space=pl.ANY),
                  vmem, vmem, vmem, vmem, vmem],
        out_shape=jax.ShapeDtypeStruct((1,), jnp.float32),
        scratch_shapes=[
            pltpu.VMEM((nout, kdim), jnp.float32),
            pltpu.VMEM((1, kdim), jnp.float32),
            pltpu.SemaphoreType.DMA((_NCHUNK,)),
        ],
    )(x, adj, W1, b1, W2, b2, fc1_w, fc1_b, fc2_w, fc2_b, fc3_w, fc3_b)

    return y


# speedup vs baseline: 2.1441x; 1.0019x over previous
import jax
import jax.numpy as jnp
from jax.experimental import pallas as pl
from jax.experimental.pallas import tpu as pltpu

_DN = (((1,), (1,)), ((), ()))
_NCHUNK = 4


def _body(x_ref, adj_ref, w1_ref, b1_ref, w2_ref, b2_ref, fc1w_hbm,
          fc1b_ref, fc2w_ref, fc2b_ref, fc3w_ref, fc3b_ref, out_ref,
          wbuf, flat_s, sems):
    nout, kdim = wbuf.shape
    chunk = kdim // _NCHUNK
    for k in range(_NCHUNK):
        sl = pl.ds(k * chunk, chunk)
        pltpu.make_async_copy(fc1w_hbm.at[:, sl], wbuf.at[:, sl],
                              sems.at[k]).start()

    s1 = jnp.dot(x_ref[...], w1_ref[...], preferred_element_type=jnp.float32)
    h1 = jax.nn.relu(
        jnp.dot(adj_ref[...], s1, preferred_element_type=jnp.float32)
        + b1_ref[...].reshape(1, -1)
    )
    s2 = jnp.dot(h1, w2_ref[...], preferred_element_type=jnp.float32)
    h2 = jax.nn.relu(
        jnp.dot(adj_ref[...], s2, preferred_element_type=jnp.float32)
        + b2_ref[...].reshape(1, -1)
    )
    n, nclass = h2.shape
    for r in range(n):
        flat_s[0:1, r * nclass:(r + 1) * nclass] = h2[r:r + 1, :]

    acc = jnp.zeros((nout, 128), jnp.float32)
    for k in range(_NCHUNK):
        sl = pl.ds(k * chunk, chunk)
        pltpu.make_async_copy(fc1w_hbm.at[:, sl], wbuf.at[:, sl],
                              sems.at[k]).wait()
        t = wbuf[:, sl] * flat_s[0:1, k * chunk:(k + 1) * chunk]
        for g in range(chunk // 128):
            acc = acc + t[:, g * 128:(g + 1) * 128]

    a1 = jax.nn.relu(acc.sum(axis=1).reshape(1, nout)
                     + fc1b_ref[...].reshape(1, -1))
    a2 = jax.nn.relu(
        jax.lax.dot_general(a1, fc2w_ref[...], _DN,
                            preferred_element_type=jnp.float32)
        + fc2b_ref[...].reshape(1, -1)
    )
    a3 = (jnp.sum(a2 * fc3w_ref[...], axis=1, keepdims=True)
          + fc3b_ref[...].reshape(1, -1))
    out_ref[...] = jax.nn.sigmoid(a3).reshape(1)


def kernel(x, adj, W1, b1, W2, b2, fc1_w, fc1_b, fc2_w, fc2_b, fc3_w, fc3_b):
    nout, kdim = fc1_w.shape
    vmem = pl.BlockSpec(memory_space=pltpu.MemorySpace.VMEM)

    y = pl.pallas_call(
        _body,
        in_specs=[vmem, vmem, vmem, vmem, vmem, vmem,
                  pl.BlockSpec(memory_space=pl.ANY),
                  vmem, vmem, vmem, vmem, vmem],
        out_shape=jax.ShapeDtypeStruct((1,), jnp.float32),
        scratch_shapes=[
            pltpu.VMEM((nout, kdim), jnp.float32),
            pltpu.VMEM((1, kdim), jnp.float32),
            pltpu.SemaphoreType.DMA((_NCHUNK,)),
        ],
    )(x, adj, W1, b1, W2, b2, fc1_w, fc1_b, fc2_w, fc2_b, fc3_w, fc3_b)

    return y
